# baseline (speedup 1.0000x reference)
import jax
import jax.numpy as jnp
from jax.experimental import pallas as pl

E = 64
K = 8
DI1 = 64
DI2 = 32
DO1 = 64
DO2 = 32
DIN = DI1 * DI2
DOUT = DO1 * DO2

BLK = 16


def _topk_weights(logits):
    cur = logits
    top1 = jnp.max(cur, axis=-1, keepdims=True)
    wacc = jnp.zeros_like(logits)
    denom = jnp.zeros_like(top1)
    iota = jax.lax.broadcasted_iota(jnp.int32, logits.shape, 1)
    for _ in range(K):
        m = jnp.max(cur, axis=-1, keepdims=True)
        sel = cur == m
        midx = jnp.min(jnp.where(sel, iota, E), axis=-1, keepdims=True)
        first = iota == midx
        ev = jnp.exp(m - top1)
        wacc = wacc + jnp.where(first, ev, 0.0)
        denom = denom + ev
        cur = jnp.where(first, -jnp.inf, cur)
    return wacc / denom


def _moe_kernel(x_ref, x2_ref, wrt_ref, sb_ref, sa_ref, tile_ref, sc_ref,
                bias_ref, out_ref):
    m = x_ref.shape[0]
    xb = x_ref[...]

    logits = jnp.dot(xb, wrt_ref[...], preferred_element_type=jnp.float32)
    w = _topk_weights(logits)
    wrow = jnp.dot(w, tile_ref[...],
                   preferred_element_type=jnp.float32)

    x2 = x2_ref[...].reshape(DI1 * m, DI2)
    v = jnp.dot(x2.astype(jnp.bfloat16), sb_ref[...],
                preferred_element_type=jnp.float32)

    vw = (v.reshape(DI1, m, DO2 * E) * wrow[None]).astype(jnp.bfloat16)

    vt = (vw.reshape(DI1, m, DO2, E).transpose(2, 1, 0, 3)
          .reshape(DO2 * m, DI1 * E))
    out = jnp.dot(vt, sa_ref[...], preferred_element_type=jnp.float32)

    res = out.reshape(DO2, m, DO1).transpose(1, 2, 0)
    out_ref[...] = res * sc_ref[0, 0] + bias_ref[...]


@jax.jit
def _run(xf, x2h, wrt, sb, sa, tile, scale2, bias3):
    n = xf.shape[0]
    grid = (n // BLK,)
    return pl.pallas_call(
        _moe_kernel,
        grid=grid,
        in_specs=[
            pl.BlockSpec((BLK, DIN), lambda i: (i, 0)),
            pl.BlockSpec((DI1, BLK, DI2), lambda i: (0, i, 0)),
            pl.BlockSpec((DIN, E), lambda i: (0, 0)),
            pl.BlockSpec((DI2, DO2 * E), lambda i: (0, 0)),
            pl.BlockSpec((DI1 * E, DO1), lambda i: (0, 0)),
            pl.BlockSpec((E, DO2 * E), lambda i: (0, 0)),
            pl.BlockSpec((1, 1), lambda i: (0, 0)),
            pl.BlockSpec((1, DO1, DO2), lambda i: (0, 0, 0)),
        ],
        out_specs=pl.BlockSpec((BLK, DO1, DO2), lambda i: (i, 0, 0)),
        out_shape=jax.ShapeDtypeStruct((n, DO1, DO2), jnp.float32),
    )(xf, x2h, wrt, sb, sa, tile, scale2, bias3)


def kernel(x, Wr, A, B, scale, bias):
    orig_shape = x.shape
    xf = x.reshape(-1, DIN)
    x2h = xf.reshape(-1, DI1, DI2).transpose(1, 0, 2)
    wrt = Wr.T
    sb = B.transpose(2, 1, 0).reshape(DI2, DO2 * E).astype(jnp.bfloat16)
    sa = A.transpose(2, 0, 1).reshape(DI1 * E, DO1).astype(jnp.bfloat16)
    tile = (jnp.arange(E)[:, None]
            == jnp.arange(DO2 * E)[None, :] % E).astype(jnp.float32)
    out = _run(xf, x2h, wrt, sb, sa, tile,
               scale.reshape(1, 1), bias.reshape(1, DO1, DO2))
    out = out.reshape(*orig_shape[:-1], DOUT)
    aux_loss = jnp.asarray(0.0, dtype=x.dtype)
    return (out, aux_loss)
